# Initial kernel scaffold; baseline (speedup 1.0000x reference)
import jax
import jax.numpy as jnp
from jax.experimental import pallas as pl


def kernel(ch_vals, edge_src, edge_dst, params):
    raise NotImplementedError("write your pallas kernel here")


# speedup vs baseline: 9.0056x; 9.0056x over previous
import functools

import jax
import jax.numpy as jnp
from jax import lax
from jax.experimental import pallas as pl
from jax.experimental.pallas import tpu as pltpu
from jax.experimental.pallas import tpu_sc as plsc

N = 10000
B = 128
E = 320000
NC, NS, L = 2, 16, 16
W = NC * NS
BLK = 128
NBLK = E // BLK
BLK_PER_W = -(-NBLK // W)
STRIPE = 624
STRIPE_LAST = N - STRIPE * (NS - 1)
GRID = 10


def _exp_body(x_ref, o_ref):
    o_ref[...] = jnp.exp(x_ref[...])


def _log_body(p_ref, o_ref):
    s = p_ref[0] + p_ref[1]
    o_ref[...] = jnp.log(jnp.maximum(s, 1e-30))


def _sc_body(ev, src, dst, p, zeros, out, src_v, dst_v, p_v, rows_v, s_sh, sem):
    cid = lax.axis_index("c")
    sid = lax.axis_index("s")
    wid = cid * NS + sid
    r0 = sid * STRIPE

    @pl.when(sid < NS - 1)
    def _():
        pltpu.sync_copy(zeros.at[pl.ds(r0, STRIPE)],
                        s_sh.at[pl.ds(r0, STRIPE)])

    @pl.when(sid == NS - 1)
    def _():
        pltpu.sync_copy(zeros.at[pl.ds(r0, STRIPE_LAST)],
                        s_sh.at[pl.ds(r0, STRIPE_LAST)])

    plsc.subcore_barrier()

    def do_block(t, carry):
        blk = wid + t * W

        @pl.when(blk < NBLK)
        def _():
            e0 = blk * BLK
            pltpu.sync_copy(src.at[pl.ds(e0, BLK)], src_v)
            pltpu.sync_copy(dst.at[pl.ds(e0, BLK)], dst_v)
            pltpu.sync_copy(p.at[pl.ds(e0, BLK)], p_v)
            pltpu.async_copy(ev.at[src_v], rows_v, sem).wait()

            def mul_group(g, c):
                p16 = p_v[pl.ds(g * L, L)]
                for k in range(L):
                    ps = jnp.full((L,), p16[k], jnp.float32)
                    row = g * L + k
                    for j in range(B // L):
                        sl = (row, pl.ds(j * L, L))
                        rows_v[sl] = rows_v[sl] * ps
                return c

            lax.fori_loop(0, BLK // L, mul_group, 0)
            pltpu.sync_copy(rows_v, s_sh.at[dst_v], add=True)

        return carry

    lax.fori_loop(0, BLK_PER_W, do_block, 0)
    plsc.subcore_barrier()

    @pl.when(sid < NS - 1)
    def _():
        pltpu.sync_copy(s_sh.at[pl.ds(r0, STRIPE)],
                        out.at[cid, pl.ds(r0, STRIPE)])

    @pl.when(sid == NS - 1)
    def _():
        pltpu.sync_copy(s_sh.at[pl.ds(r0, STRIPE_LAST)],
                        out.at[cid, pl.ds(r0, STRIPE_LAST)])


def kernel(ch_vals, edge_src, edge_dst, params):
    ev = pl.pallas_call(
        _exp_body,
        grid=(GRID,),
        in_specs=[pl.BlockSpec((N // GRID, B), lambda i: (i, 0))],
        out_specs=pl.BlockSpec((N // GRID, B), lambda i: (i, 0)),
        out_shape=jax.ShapeDtypeStruct((N, B), jnp.float32),
    )(ch_vals)

    zeros = jnp.zeros((N, B), jnp.float32)

    sc = pl.kernel(
        _sc_body,
        out_type=jax.ShapeDtypeStruct((NC, N, B), jnp.float32),
        mesh=plsc.VectorSubcoreMesh(core_axis_name="c", subcore_axis_name="s"),
        scratch_types=[
            pltpu.VMEM((BLK,), jnp.int32),
            pltpu.VMEM((BLK,), jnp.int32),
            pltpu.VMEM((BLK,), jnp.float32),
            pltpu.VMEM((BLK, B), jnp.float32),
            pltpu.VMEM_SHARED((N, B), jnp.float32),
            pltpu.SemaphoreType.DMA,
        ],
    )
    partial = sc(ev, edge_src, edge_dst, params, zeros)

    out = pl.pallas_call(
        _log_body,
        grid=(GRID,),
        in_specs=[pl.BlockSpec((NC, N // GRID, B), lambda i: (0, i, 0))],
        out_specs=pl.BlockSpec((N // GRID, B), lambda i: (i, 0)),
        out_shape=jax.ShapeDtypeStruct((N, B), jnp.float32),
    )(partial)
    return out
